# Initial kernel scaffold; baseline (speedup 1.0000x reference)
import jax
import jax.numpy as jnp
from jax.experimental import pallas as pl


def kernel(x, edge_index, W1, b1, W2, b2):
    raise NotImplementedError("write your pallas kernel here")


# speedup vs baseline: 14.3962x; 14.3962x over previous
import functools

import jax
import jax.numpy as jnp
from jax import lax
from jax.experimental import pallas as pl
from jax.experimental.pallas import tpu as pltpu
from jax.experimental.pallas import tpu_sc as plsc

NC = 2
NS = 16
NW = NC * NS
CHUNK = 80


def _pad_nodes(n):
    q = NS * 8
    return (n + q - 1) // q * q


def _sc_mesh():
    return plsc.VectorSubcoreMesh(core_axis_name="c", subcore_axis_name="s")


def _sc_degree(dst, zeros16, n_pad):
    n_edges = dst.shape[0]
    epw = n_edges // NW
    steps = epw // CHUNK
    rpt = n_pad // NS

    @functools.partial(
        pl.kernel,
        out_type=jax.ShapeDtypeStruct((NC, n_pad, 16), jnp.float32),
        mesh=_sc_mesh(),
        scratch_types=[
            pltpu.VMEM((CHUNK,), jnp.int32),
            pltpu.VMEM((CHUNK, 16), jnp.float32),
            pltpu.VMEM_SHARED((n_pad, 16), jnp.float32),
            pltpu.SemaphoreType.DMA,
        ],
    )
    def k(dst_hbm, z_hbm, out_hbm, idx_v, ones_v, acc_sh, sem):
        c = lax.axis_index("c")
        s = lax.axis_index("s")
        wid = s * NC + c
        pltpu.sync_copy(z_hbm.at[pl.ds(s * rpt, rpt)],
                        acc_sh.at[pl.ds(s * rpt, rpt)])

        @pl.loop(0, CHUNK)
        def _(r):
            ones_v[r, :] = jnp.ones((16,), jnp.float32)

        plsc.subcore_barrier()
        base = wid * epw

        @pl.loop(0, steps)
        def _(i):
            pltpu.sync_copy(dst_hbm.at[pl.ds(base + i * CHUNK, CHUNK)], idx_v)
            pltpu.sync_copy(ones_v, acc_sh.at[idx_v], add=True)

        plsc.subcore_barrier()
        pltpu.sync_copy(acc_sh.at[pl.ds(s * rpt, rpt)],
                        out_hbm.at[c, pl.ds(s * rpt, rpt)])

    return k(dst, zeros16)


def _sc_scatter(g, src, dst, zeros, n_pad):
    n_edges = src.shape[0]
    d = g.shape[1]
    epw = n_edges // NW
    steps = epw // CHUNK
    rpt = n_pad // NS

    @functools.partial(
        pl.kernel,
        out_type=jax.ShapeDtypeStruct((NC, n_pad, d), jnp.float32),
        mesh=_sc_mesh(),
        scratch_types=[
            pltpu.VMEM((CHUNK,), jnp.int32),
            pltpu.VMEM((CHUNK,), jnp.int32),
            pltpu.VMEM((CHUNK, d), jnp.float32),
            pltpu.VMEM_SHARED((n_pad, d), jnp.float32),
            pltpu.SemaphoreType.DMA,
        ],
    )
    def k(g_hbm, src_hbm, dst_hbm, z_hbm, out_hbm, sidx, didx, rows, acc_sh, sem):
        c = lax.axis_index("c")
        s = lax.axis_index("s")
        wid = s * NC + c
        pltpu.sync_copy(z_hbm.at[pl.ds(s * rpt, rpt)],
                        acc_sh.at[pl.ds(s * rpt, rpt)])
        plsc.subcore_barrier()
        base = wid * epw

        @pl.loop(0, steps)
        def _(i):
            off = base + i * CHUNK
            pltpu.sync_copy(src_hbm.at[pl.ds(off, CHUNK)], sidx)
            pltpu.sync_copy(dst_hbm.at[pl.ds(off, CHUNK)], didx)
            pltpu.async_copy(g_hbm.at[sidx], rows, sem).wait()
            pltpu.sync_copy(rows, acc_sh.at[didx], add=True)

        plsc.subcore_barrier()
        pltpu.sync_copy(acc_sh.at[pl.ds(s * rpt, rpt)],
                        out_hbm.at[c, pl.ds(s * rpt, rpt)])

    return k(g, src, dst, zeros)


_ROWS = 1000


def _tc_prep(x, degp, n_nodes):
    d_in = x.shape[1]
    grid = n_nodes // _ROWS

    def body(x_ref, d0_ref, d1_ref, g1_ref, dv_ref):
        deg = 1.0 + d0_ref[0, :, 0:1] + d1_ref[0, :, 0:1]
        dinv = lax.rsqrt(deg)
        g1_ref[...] = x_ref[...] * dinv
        dv_ref[...] = jnp.broadcast_to(dinv, dv_ref.shape)

    return pl.pallas_call(
        body,
        grid=(grid,),
        in_specs=[
            pl.BlockSpec((_ROWS, d_in), lambda i: (i, 0)),
            pl.BlockSpec((1, _ROWS, 16), lambda i: (0, i, 0)),
            pl.BlockSpec((1, _ROWS, 16), lambda i: (1, i, 0)),
        ],
        out_specs=[
            pl.BlockSpec((_ROWS, d_in), lambda i: (i, 0)),
            pl.BlockSpec((_ROWS, 128), lambda i: (i, 0)),
        ],
        out_shape=[
            jax.ShapeDtypeStruct((n_nodes, d_in), jnp.float32),
            jax.ShapeDtypeStruct((n_nodes, 128), jnp.float32),
        ],
    )(x, degp, degp)


def _tc_mid(p1, g1, dv, w1, b1, w2, n_nodes):
    d_in = g1.shape[1]
    d_hid = w1.shape[1]
    d_out = w2.shape[1]
    grid = n_nodes // _ROWS

    def body(p0_ref, p1_ref, g1_ref, dv_ref, w1_ref, b1_ref, w2_ref, g2_ref):
        dinv = dv_ref[:, 0:1]
        s1 = dinv * (p0_ref[0] + p1_ref[0] + g1_ref[...])
        h1 = jnp.dot(s1, w1_ref[...], preferred_element_type=jnp.float32)
        h1 = jnp.maximum(h1 + b1_ref[...], 0.0)
        m2 = jnp.dot(h1, w2_ref[...], preferred_element_type=jnp.float32)
        g2_ref[...] = dinv * m2

    return pl.pallas_call(
        body,
        grid=(grid,),
        in_specs=[
            pl.BlockSpec((1, _ROWS, d_in), lambda i: (0, i, 0)),
            pl.BlockSpec((1, _ROWS, d_in), lambda i: (1, i, 0)),
            pl.BlockSpec((_ROWS, d_in), lambda i: (i, 0)),
            pl.BlockSpec((_ROWS, 128), lambda i: (i, 0)),
            pl.BlockSpec((d_in, d_hid), lambda i: (0, 0)),
            pl.BlockSpec((1, d_hid), lambda i: (0, 0)),
            pl.BlockSpec((d_hid, d_out), lambda i: (0, 0)),
        ],
        out_specs=pl.BlockSpec((_ROWS, d_out), lambda i: (i, 0)),
        out_shape=jax.ShapeDtypeStruct((n_nodes, d_out), jnp.float32),
    )(p1, p1, g1, dv, w1, b1, w2)


def _tc_final(p2, g2, dv, b2, n_nodes):
    d_out = g2.shape[1]
    grid = n_nodes // _ROWS

    def body(p0_ref, p1_ref, g2_ref, dv_ref, b2_ref, z_ref):
        dinv = dv_ref[:, 0:1]
        z_ref[...] = dinv * (p0_ref[0] + p1_ref[0] + g2_ref[...]) + b2_ref[...]

    return pl.pallas_call(
        body,
        grid=(grid,),
        in_specs=[
            pl.BlockSpec((1, _ROWS, d_out), lambda i: (0, i, 0)),
            pl.BlockSpec((1, _ROWS, d_out), lambda i: (1, i, 0)),
            pl.BlockSpec((_ROWS, d_out), lambda i: (i, 0)),
            pl.BlockSpec((_ROWS, 128), lambda i: (i, 0)),
            pl.BlockSpec((1, d_out), lambda i: (0, 0)),
        ],
        out_specs=pl.BlockSpec((_ROWS, d_out), lambda i: (i, 0)),
        out_shape=jax.ShapeDtypeStruct((n_nodes, d_out), jnp.float32),
    )(p2, p2, g2, dv, b2)


def kernel(x, edge_index, W1, b1, W2, b2):
    n = x.shape[0]
    n_pad = _pad_nodes(n)
    ei = edge_index.astype(jnp.int32)
    src, dst = ei[0], ei[1]
    zeros16 = jnp.zeros((n_pad, 16), jnp.float32)
    zeros_d = jnp.zeros((n_pad, x.shape[1]), jnp.float32)

    degp = _sc_degree(dst, zeros16, n_pad)
    g1, dv = _tc_prep(x, degp, n)
    p1 = _sc_scatter(g1, src, dst, zeros_d, n_pad)
    g2 = _tc_mid(p1, g1, dv, W1, b1.reshape(1, -1), W2, n)
    p2 = _sc_scatter(g2, src, dst, zeros_d, n_pad)
    z = _tc_final(p2, g2, dv, b2.reshape(1, -1), n)
    return z


# speedup vs baseline: 23.4601x; 1.6296x over previous
import functools

import jax
import jax.numpy as jnp
from jax import lax
from jax.experimental import pallas as pl
from jax.experimental.pallas import tpu as pltpu
from jax.experimental.pallas import tpu_sc as plsc

NC = 2
NS = 16
NW = NC * NS
CHUNK = 80


def _pad_nodes(n):
    q = NS * 8
    return (n + q - 1) // q * q


def _sc_mesh():
    return plsc.VectorSubcoreMesh(core_axis_name="c", subcore_axis_name="s")


def _sc_degree(dst3, zeros16, n_pad):
    steps = dst3.shape[1]
    rpt = n_pad // NS

    @functools.partial(
        pl.kernel,
        out_type=jax.ShapeDtypeStruct((NC, n_pad, 16), jnp.float32),
        mesh=_sc_mesh(),
        scratch_types=[
            pltpu.VMEM((CHUNK,), jnp.int32),
            pltpu.VMEM((CHUNK, 16), jnp.float32),
            pltpu.VMEM_SHARED((n_pad, 16), jnp.float32),
            pltpu.SemaphoreType.DMA,
        ],
    )
    def k(dst_hbm, z_hbm, out_hbm, idx_v, ones_v, acc_sh, sem):
        c = lax.axis_index("c")
        s = lax.axis_index("s")
        wid = s * NC + c
        pltpu.sync_copy(z_hbm.at[pl.ds(s * rpt, rpt)],
                        acc_sh.at[pl.ds(s * rpt, rpt)])

        @pl.loop(0, CHUNK)
        def _(r):
            ones_v[r, :] = jnp.ones((16,), jnp.float32)

        plsc.subcore_barrier()

        @pl.loop(0, steps)
        def _(i):
            pltpu.sync_copy(dst_hbm.at[wid, i], idx_v)
            pltpu.sync_copy(ones_v, acc_sh.at[idx_v], add=True)

        plsc.subcore_barrier()
        pltpu.sync_copy(acc_sh.at[pl.ds(s * rpt, rpt)],
                        out_hbm.at[c, pl.ds(s * rpt, rpt)])

    return k(dst3, zeros16)


def _sc_scatter(g, src3, dst3, zeros, n_pad):
    d = g.shape[1]
    steps = src3.shape[1]
    pairs = (steps - 1) // 2
    rpt = n_pad // NS

    @functools.partial(
        pl.kernel,
        out_type=jax.ShapeDtypeStruct((NC, n_pad, d), jnp.float32),
        mesh=_sc_mesh(),
        scratch_types=[
            pltpu.VMEM((CHUNK,), jnp.int32),
            pltpu.VMEM((CHUNK,), jnp.int32),
            pltpu.VMEM((CHUNK,), jnp.int32),
            pltpu.VMEM((CHUNK,), jnp.int32),
            pltpu.VMEM((CHUNK, d), jnp.float32),
            pltpu.VMEM((CHUNK, d), jnp.float32),
            pltpu.VMEM_SHARED((n_pad, d), jnp.float32),
            pltpu.SemaphoreType.DMA,
            pltpu.SemaphoreType.DMA,
            pltpu.SemaphoreType.DMA,
            pltpu.SemaphoreType.DMA,
        ],
    )
    def k(g_hbm, src_hbm, dst_hbm, z_hbm, out_hbm,
          sidx_a, didx_a, sidx_b, didx_b, rows_a, rows_b, acc_sh,
          sem_a, sem_b, sem_ia, sem_ib):
        c = lax.axis_index("c")
        s = lax.axis_index("s")
        wid = s * NC + c
        pltpu.sync_copy(z_hbm.at[pl.ds(s * rpt, rpt)],
                        acc_sh.at[pl.ds(s * rpt, rpt)])

        def ix_start(i, sidx, didx, sem):
            pltpu.make_async_copy(src_hbm.at[wid, i], sidx, sem).start()
            pltpu.make_async_copy(dst_hbm.at[wid, i], didx, sem).start()

        def ix_wait(i, sidx, didx, sem):
            pltpu.make_async_copy(src_hbm.at[wid, i], sidx, sem).wait()
            pltpu.make_async_copy(dst_hbm.at[wid, i], didx, sem).wait()

        def g_start(sidx, rows, sem):
            pltpu.make_async_copy(g_hbm.at[sidx], rows, sem).start()

        def g_wait(sidx, rows, sem):
            pltpu.make_async_copy(g_hbm.at[sidx], rows, sem).wait()

        def s_add(didx, rows):
            pltpu.sync_copy(rows, acc_sh.at[didx], add=True)

        pltpu.sync_copy(src_hbm.at[wid, 0], sidx_a)
        pltpu.sync_copy(dst_hbm.at[wid, 0], didx_a)
        plsc.subcore_barrier()
        g_start(sidx_a, rows_a, sem_a)
        ix_start(1, sidx_b, didx_b, sem_ib)

        @pl.loop(0, pairs)
        def _(p):
            i = 2 * p
            g_wait(sidx_a, rows_a, sem_a)
            ix_wait(i + 1, sidx_b, didx_b, sem_ib)
            g_start(sidx_b, rows_b, sem_b)
            s_add(didx_a, rows_a)
            ix_start(i + 2, sidx_a, didx_a, sem_ia)
            g_wait(sidx_b, rows_b, sem_b)
            ix_wait(i + 2, sidx_a, didx_a, sem_ia)
            g_start(sidx_a, rows_a, sem_a)
            s_add(didx_b, rows_b)

            @pl.when(i + 3 < steps)
            def _():
                ix_start(i + 3, sidx_b, didx_b, sem_ib)

        g_wait(sidx_a, rows_a, sem_a)
        s_add(didx_a, rows_a)

        plsc.subcore_barrier()
        pltpu.sync_copy(acc_sh.at[pl.ds(s * rpt, rpt)],
                        out_hbm.at[c, pl.ds(s * rpt, rpt)])

    return k(g, src3, dst3, zeros)


_ROWS = 1000


def _tc_prep(x, degp, n_nodes):
    d_in = x.shape[1]
    grid = n_nodes // _ROWS

    def body(x_ref, d0_ref, d1_ref, g1_ref, dv_ref):
        deg = 1.0 + d0_ref[0, :, 0:1] + d1_ref[0, :, 0:1]
        dinv = lax.rsqrt(deg)
        g1_ref[...] = x_ref[...] * dinv
        dv_ref[...] = jnp.broadcast_to(dinv, dv_ref.shape)

    return pl.pallas_call(
        body,
        grid=(grid,),
        in_specs=[
            pl.BlockSpec((_ROWS, d_in), lambda i: (i, 0)),
            pl.BlockSpec((1, _ROWS, 16), lambda i: (0, i, 0)),
            pl.BlockSpec((1, _ROWS, 16), lambda i: (1, i, 0)),
        ],
        out_specs=[
            pl.BlockSpec((_ROWS, d_in), lambda i: (i, 0)),
            pl.BlockSpec((_ROWS, 128), lambda i: (i, 0)),
        ],
        out_shape=[
            jax.ShapeDtypeStruct((n_nodes, d_in), jnp.float32),
            jax.ShapeDtypeStruct((n_nodes, 128), jnp.float32),
        ],
    )(x, degp, degp)


def _tc_mid(p1, g1, dv, w1, b1, w2, n_nodes):
    d_in = g1.shape[1]
    d_hid = w1.shape[1]
    d_out = w2.shape[1]
    grid = n_nodes // _ROWS

    def body(p0_ref, p1_ref, g1_ref, dv_ref, w1_ref, b1_ref, w2_ref, g2_ref):
        dinv = dv_ref[:, 0:1]
        s1 = dinv * (p0_ref[0] + p1_ref[0] + g1_ref[...])
        h1 = jnp.dot(s1, w1_ref[...], preferred_element_type=jnp.float32)
        h1 = jnp.maximum(h1 + b1_ref[...], 0.0)
        m2 = jnp.dot(h1, w2_ref[...], preferred_element_type=jnp.float32)
        g2_ref[...] = dinv * m2

    return pl.pallas_call(
        body,
        grid=(grid,),
        in_specs=[
            pl.BlockSpec((1, _ROWS, d_in), lambda i: (0, i, 0)),
            pl.BlockSpec((1, _ROWS, d_in), lambda i: (1, i, 0)),
            pl.BlockSpec((_ROWS, d_in), lambda i: (i, 0)),
            pl.BlockSpec((_ROWS, 128), lambda i: (i, 0)),
            pl.BlockSpec((d_in, d_hid), lambda i: (0, 0)),
            pl.BlockSpec((1, d_hid), lambda i: (0, 0)),
            pl.BlockSpec((d_hid, d_out), lambda i: (0, 0)),
        ],
        out_specs=pl.BlockSpec((_ROWS, d_out), lambda i: (i, 0)),
        out_shape=jax.ShapeDtypeStruct((n_nodes, d_out), jnp.float32),
    )(p1, p1, g1, dv, w1, b1, w2)


def _tc_final(p2, g2, dv, b2, n_nodes):
    d_out = g2.shape[1]
    grid = n_nodes // _ROWS

    def body(p0_ref, p1_ref, g2_ref, dv_ref, b2_ref, z_ref):
        dinv = dv_ref[:, 0:1]
        z_ref[...] = dinv * (p0_ref[0] + p1_ref[0] + g2_ref[...]) + b2_ref[...]

    return pl.pallas_call(
        body,
        grid=(grid,),
        in_specs=[
            pl.BlockSpec((1, _ROWS, d_out), lambda i: (0, i, 0)),
            pl.BlockSpec((1, _ROWS, d_out), lambda i: (1, i, 0)),
            pl.BlockSpec((_ROWS, d_out), lambda i: (i, 0)),
            pl.BlockSpec((_ROWS, 128), lambda i: (i, 0)),
            pl.BlockSpec((1, d_out), lambda i: (0, 0)),
        ],
        out_specs=pl.BlockSpec((_ROWS, d_out), lambda i: (i, 0)),
        out_shape=jax.ShapeDtypeStruct((n_nodes, d_out), jnp.float32),
    )(p2, p2, g2, dv, b2)


def kernel(x, edge_index, W1, b1, W2, b2):
    n = x.shape[0]
    n_pad = _pad_nodes(n)
    ei = edge_index.astype(jnp.int32)
    epw = ei.shape[1] // NW
    steps = epw // CHUNK
    src3 = ei[0].reshape(NW, steps, CHUNK)
    dst3 = ei[1].reshape(NW, steps, CHUNK)
    zeros16 = jnp.zeros((n_pad, 16), jnp.float32)
    zeros_d = jnp.zeros((n_pad, x.shape[1]), jnp.float32)

    degp = _sc_degree(dst3, zeros16, n_pad)
    g1, dv = _tc_prep(x, degp, n)
    p1 = _sc_scatter(g1, src3, dst3, zeros_d, n_pad)
    g2 = _tc_mid(p1, g1, dv, W1, b1.reshape(1, -1), W2, n)
    p2 = _sc_scatter(g2, src3, dst3, zeros_d, n_pad)
    z = _tc_final(p2, g2, dv, b2.reshape(1, -1), n)
    return z


# speedup vs baseline: 24.1975x; 1.0314x over previous
import functools

import jax
import jax.numpy as jnp
from jax import lax
from jax.experimental import pallas as pl
from jax.experimental.pallas import tpu as pltpu
from jax.experimental.pallas import tpu_sc as plsc

NC = 2
NS = 16
NW = NC * NS
CHUNK = 80


def _pad_nodes(n):
    q = NS * 8
    return (n + q - 1) // q * q


def _sc_mesh():
    return plsc.VectorSubcoreMesh(core_axis_name="c", subcore_axis_name="s")


def _sc_degree(dst3, zeros16, n_pad):
    steps = dst3.shape[1]
    rpt = n_pad // NS

    @functools.partial(
        pl.kernel,
        out_type=jax.ShapeDtypeStruct((NC, n_pad, 16), jnp.float32),
        mesh=_sc_mesh(),
        scratch_types=[
            pltpu.VMEM((CHUNK,), jnp.int32),
            pltpu.VMEM((CHUNK,), jnp.int32),
            pltpu.VMEM((CHUNK, 16), jnp.float32),
            pltpu.VMEM_SHARED((n_pad, 16), jnp.float32),
            pltpu.SemaphoreType.DMA,
            pltpu.SemaphoreType.DMA,
        ],
    )
    def k(dst_hbm, z_hbm, out_hbm, idx_a, idx_b, ones_v, acc_sh,
          sem_ia, sem_ib):
        c = lax.axis_index("c")
        s = lax.axis_index("s")
        wid = s * NC + c
        pltpu.sync_copy(z_hbm.at[pl.ds(s * rpt, rpt)],
                        acc_sh.at[pl.ds(s * rpt, rpt)])

        @pl.loop(0, CHUNK)
        def _(r):
            ones_v[r, :] = jnp.ones((16,), jnp.float32)

        def ix_start(i, idx, sem):
            pltpu.make_async_copy(dst_hbm.at[wid, i], idx, sem).start()

        def ix_wait(i, idx, sem):
            pltpu.make_async_copy(dst_hbm.at[wid, i], idx, sem).wait()

        def s_add(idx):
            pltpu.sync_copy(ones_v, acc_sh.at[idx], add=True)

        pltpu.sync_copy(dst_hbm.at[wid, 0], idx_a)
        plsc.subcore_barrier()
        ix_start(1, idx_b, sem_ib)
        pairs = (steps - 1) // 2

        @pl.loop(0, pairs)
        def _(p):
            i = 2 * p
            s_add(idx_a)
            ix_wait(i + 1, idx_b, sem_ib)
            ix_start(i + 2, idx_a, sem_ia)
            s_add(idx_b)
            ix_wait(i + 2, idx_a, sem_ia)

            @pl.when(i + 3 < steps)
            def _():
                ix_start(i + 3, idx_b, sem_ib)

        s_add(idx_a)

        plsc.subcore_barrier()
        pltpu.sync_copy(acc_sh.at[pl.ds(s * rpt, rpt)],
                        out_hbm.at[c, pl.ds(s * rpt, rpt)])

    return k(dst3, zeros16)


def _sc_scatter(g, src3, dst3, zeros, n_pad):
    d = g.shape[1]
    steps = src3.shape[1]
    pairs = (steps - 1) // 2
    rpt = n_pad // NS

    @functools.partial(
        pl.kernel,
        out_type=jax.ShapeDtypeStruct((NC, n_pad, d), jnp.float32),
        mesh=_sc_mesh(),
        scratch_types=[
            pltpu.VMEM((CHUNK,), jnp.int32),
            pltpu.VMEM((CHUNK,), jnp.int32),
            pltpu.VMEM((CHUNK,), jnp.int32),
            pltpu.VMEM((CHUNK,), jnp.int32),
            pltpu.VMEM((CHUNK, d), jnp.float32),
            pltpu.VMEM((CHUNK, d), jnp.float32),
            pltpu.VMEM_SHARED((n_pad, d), jnp.float32),
            pltpu.SemaphoreType.DMA,
            pltpu.SemaphoreType.DMA,
            pltpu.SemaphoreType.DMA,
            pltpu.SemaphoreType.DMA,
        ],
    )
    def k(g_hbm, src_hbm, dst_hbm, z_hbm, out_hbm,
          sidx_a, didx_a, sidx_b, didx_b, rows_a, rows_b, acc_sh,
          sem_a, sem_b, sem_ia, sem_ib):
        c = lax.axis_index("c")
        s = lax.axis_index("s")
        wid = s * NC + c
        pltpu.sync_copy(z_hbm.at[pl.ds(s * rpt, rpt)],
                        acc_sh.at[pl.ds(s * rpt, rpt)])

        def ix_start(i, sidx, didx, sem):
            pltpu.make_async_copy(src_hbm.at[wid, i], sidx, sem).start()
            pltpu.make_async_copy(dst_hbm.at[wid, i], didx, sem).start()

        def ix_wait(i, sidx, didx, sem):
            pltpu.make_async_copy(src_hbm.at[wid, i], sidx, sem).wait()
            pltpu.make_async_copy(dst_hbm.at[wid, i], didx, sem).wait()

        def g_start(sidx, rows, sem):
            pltpu.make_async_copy(g_hbm.at[sidx], rows, sem).start()

        def g_wait(sidx, rows, sem):
            pltpu.make_async_copy(g_hbm.at[sidx], rows, sem).wait()

        def s_add(didx, rows):
            pltpu.sync_copy(rows, acc_sh.at[didx], add=True)

        pltpu.sync_copy(src_hbm.at[wid, 0], sidx_a)
        pltpu.sync_copy(dst_hbm.at[wid, 0], didx_a)
        plsc.subcore_barrier()
        g_start(sidx_a, rows_a, sem_a)
        ix_start(1, sidx_b, didx_b, sem_ib)

        @pl.loop(0, pairs)
        def _(p):
            i = 2 * p
            g_wait(sidx_a, rows_a, sem_a)
            ix_wait(i + 1, sidx_b, didx_b, sem_ib)
            g_start(sidx_b, rows_b, sem_b)
            s_add(didx_a, rows_a)
            ix_start(i + 2, sidx_a, didx_a, sem_ia)
            g_wait(sidx_b, rows_b, sem_b)
            ix_wait(i + 2, sidx_a, didx_a, sem_ia)
            g_start(sidx_a, rows_a, sem_a)
            s_add(didx_b, rows_b)

            @pl.when(i + 3 < steps)
            def _():
                ix_start(i + 3, sidx_b, didx_b, sem_ib)

        g_wait(sidx_a, rows_a, sem_a)
        s_add(didx_a, rows_a)

        plsc.subcore_barrier()
        pltpu.sync_copy(acc_sh.at[pl.ds(s * rpt, rpt)],
                        out_hbm.at[c, pl.ds(s * rpt, rpt)])

    return k(g, src3, dst3, zeros)


_ROWS = 1000


def _tc_prep(x, degp, n_nodes):
    d_in = x.shape[1]
    grid = n_nodes // _ROWS

    def body(x_ref, d0_ref, d1_ref, g1_ref, dv_ref):
        deg = 1.0 + d0_ref[0, :, 0:1] + d1_ref[0, :, 0:1]
        dinv = lax.rsqrt(deg)
        g1_ref[...] = x_ref[...] * dinv
        dv_ref[...] = jnp.broadcast_to(dinv, dv_ref.shape)

    return pl.pallas_call(
        body,
        grid=(grid,),
        in_specs=[
            pl.BlockSpec((_ROWS, d_in), lambda i: (i, 0)),
            pl.BlockSpec((1, _ROWS, 16), lambda i: (0, i, 0)),
            pl.BlockSpec((1, _ROWS, 16), lambda i: (1, i, 0)),
        ],
        out_specs=[
            pl.BlockSpec((_ROWS, d_in), lambda i: (i, 0)),
            pl.BlockSpec((_ROWS, 128), lambda i: (i, 0)),
        ],
        out_shape=[
            jax.ShapeDtypeStruct((n_nodes, d_in), jnp.float32),
            jax.ShapeDtypeStruct((n_nodes, 128), jnp.float32),
        ],
    )(x, degp, degp)


def _tc_mid(p1, g1, dv, w1, b1, w2, n_nodes):
    d_in = g1.shape[1]
    d_hid = w1.shape[1]
    d_out = w2.shape[1]
    grid = n_nodes // _ROWS

    def body(p0_ref, p1_ref, g1_ref, dv_ref, w1_ref, b1_ref, w2_ref, g2_ref):
        dinv = dv_ref[:, 0:1]
        s1 = dinv * (p0_ref[0] + p1_ref[0] + g1_ref[...])
        h1 = jnp.dot(s1, w1_ref[...], preferred_element_type=jnp.float32)
        h1 = jnp.maximum(h1 + b1_ref[...], 0.0)
        m2 = jnp.dot(h1, w2_ref[...], preferred_element_type=jnp.float32)
        g2_ref[...] = dinv * m2

    return pl.pallas_call(
        body,
        grid=(grid,),
        in_specs=[
            pl.BlockSpec((1, _ROWS, d_in), lambda i: (0, i, 0)),
            pl.BlockSpec((1, _ROWS, d_in), lambda i: (1, i, 0)),
            pl.BlockSpec((_ROWS, d_in), lambda i: (i, 0)),
            pl.BlockSpec((_ROWS, 128), lambda i: (i, 0)),
            pl.BlockSpec((d_in, d_hid), lambda i: (0, 0)),
            pl.BlockSpec((1, d_hid), lambda i: (0, 0)),
            pl.BlockSpec((d_hid, d_out), lambda i: (0, 0)),
        ],
        out_specs=pl.BlockSpec((_ROWS, d_out), lambda i: (i, 0)),
        out_shape=jax.ShapeDtypeStruct((n_nodes, d_out), jnp.float32),
    )(p1, p1, g1, dv, w1, b1, w2)


def _tc_final(p2, g2, dv, b2, n_nodes):
    d_out = g2.shape[1]
    grid = n_nodes // _ROWS

    def body(p0_ref, p1_ref, g2_ref, dv_ref, b2_ref, z_ref):
        dinv = dv_ref[:, 0:1]
        z_ref[...] = dinv * (p0_ref[0] + p1_ref[0] + g2_ref[...]) + b2_ref[...]

    return pl.pallas_call(
        body,
        grid=(grid,),
        in_specs=[
            pl.BlockSpec((1, _ROWS, d_out), lambda i: (0, i, 0)),
            pl.BlockSpec((1, _ROWS, d_out), lambda i: (1, i, 0)),
            pl.BlockSpec((_ROWS, d_out), lambda i: (i, 0)),
            pl.BlockSpec((_ROWS, 128), lambda i: (i, 0)),
            pl.BlockSpec((1, d_out), lambda i: (0, 0)),
        ],
        out_specs=pl.BlockSpec((_ROWS, d_out), lambda i: (i, 0)),
        out_shape=jax.ShapeDtypeStruct((n_nodes, d_out), jnp.float32),
    )(p2, p2, g2, dv, b2)


def kernel(x, edge_index, W1, b1, W2, b2):
    n = x.shape[0]
    n_pad = _pad_nodes(n)
    ei = edge_index.astype(jnp.int32)
    epw = ei.shape[1] // NW
    steps = -(-epw // CHUNK)
    if steps % 2 == 0:
        steps += 1
    pad = steps * CHUNK - epw
    src3 = jnp.pad(ei[0].reshape(NW, epw), ((0, 0), (0, pad)),
                   constant_values=0).reshape(NW, steps, CHUNK)
    dst3 = jnp.pad(ei[1].reshape(NW, epw), ((0, 0), (0, pad)),
                   constant_values=n).reshape(NW, steps, CHUNK)
    zeros16 = jnp.zeros((n_pad, 16), jnp.float32)
    zeros_d = jnp.zeros((n_pad, x.shape[1]), jnp.float32)

    degp = _sc_degree(dst3, zeros16, n_pad)
    g1, dv = _tc_prep(x, degp, n)
    p1 = _sc_scatter(g1, src3, dst3, zeros_d, n_pad)
    g2 = _tc_mid(p1, g1, dv, W1, b1.reshape(1, -1), W2, n)
    p2 = _sc_scatter(g2, src3, dst3, zeros_d, n_pad)
    z = _tc_final(p2, g2, dv, b2.reshape(1, -1), n)
    return z
